# Initial kernel scaffold; baseline (speedup 1.0000x reference)
import jax
import jax.numpy as jnp
from jax.experimental import pallas as pl


def kernel(X, eye):
    raise NotImplementedError("write your pallas kernel here")


# speedup vs baseline: 1.0064x; 1.0064x over previous
import functools

import jax
import jax.numpy as jnp
from jax import lax
from jax.experimental import pallas as pl
from jax.experimental.pallas import tpu as pltpu
from jax.experimental.pallas import tpu_sc as plsc

N_CLS = 1000
BATCH = 16384
ROWS_PER_CHUNK = 64
CHUNK_WORDS = ROWS_PER_CHUNK * N_CLS


def _build_onehot():
    info = plsc.get_sparse_core_info()
    nw = info.num_cores * info.num_subcores
    rows_per_w = BATCH // nw
    n_chunks = rows_per_w // ROWS_PER_CHUNK
    mesh = plsc.VectorSubcoreMesh(core_axis_name="c", subcore_axis_name="s")

    @functools.partial(
        pl.kernel,
        mesh=mesh,
        compiler_params=pltpu.CompilerParams(needs_layout_passes=False),
        out_type=jax.ShapeDtypeStruct((BATCH * N_CLS,), jnp.float32),
        scratch_types=[
            pltpu.VMEM((rows_per_w,), jnp.int32),
            pltpu.VMEM((CHUNK_WORDS,), jnp.float32),
            pltpu.VMEM((CHUNK_WORDS,), jnp.float32),
            pltpu.SemaphoreType.DMA,
            pltpu.SemaphoreType.DMA,
        ],
    )
    def k(x_hbm, zeros_hbm, out_hbm, idx_v, buf0, buf1, sem0, sem1):
        wid = lax.axis_index("s") * info.num_cores + lax.axis_index("c")
        row_base = wid * rows_per_w

        pltpu.sync_copy(x_hbm.at[pl.ds(row_base, rows_per_w)], idx_v)
        pltpu.sync_copy(zeros_hbm, buf0)
        pltpu.sync_copy(zeros_hbm, buf1)

        lane = lax.iota(jnp.int32, 16)
        ones16 = jnp.full((16,), 1.0, jnp.float32)
        zeros16 = jnp.zeros((16,), jnp.float32)
        bufs = (buf0, buf1)
        sems = (sem0, sem1)
        handles = [None, None]

        def flat_idx(chunk, grp):
            x16 = idx_v[pl.ds(chunk * ROWS_PER_CHUNK + grp * 16, 16)]
            lrow = grp * 16 + lane
            return lrow * N_CLS + x16

        for c in range(n_chunks):
            b = c & 1
            if handles[b] is not None:
                handles[b].wait()
                for g in range(ROWS_PER_CHUNK // 16):
                    plsc.store_scatter(bufs[b], [flat_idx(c - 2, g)], zeros16)
            for g in range(ROWS_PER_CHUNK // 16):
                plsc.store_scatter(bufs[b], [flat_idx(c, g)], ones16)
            dst = out_hbm.at[pl.ds(row_base * N_CLS + c * CHUNK_WORDS, CHUNK_WORDS)]
            handles[b] = pltpu.async_copy(bufs[b], dst, sems[b])

        handles[0].wait()
        handles[1].wait()

    return k


_onehot = _build_onehot()


def kernel(X, eye):
    del eye
    zeros_blk = jnp.zeros((CHUNK_WORDS,), jnp.float32)
    out_flat = _onehot(X, zeros_blk)
    return out_flat.reshape(BATCH, N_CLS)


# speedup vs baseline: 1.5097x; 1.5001x over previous
import functools

import jax
import jax.numpy as jnp
from jax import lax
from jax.experimental import pallas as pl
from jax.experimental.pallas import tpu as pltpu
from jax.experimental.pallas import tpu_sc as plsc

N_CLS = 1000
BATCH = 16384
ROWS_PER_CHUNK = 32
NBUF = 3


def _build_onehot():
    info = plsc.get_sparse_core_info()
    nw = info.num_cores * info.num_subcores
    rows_per_w = BATCH // nw
    n_chunks = rows_per_w // ROWS_PER_CHUNK
    mesh = plsc.VectorSubcoreMesh(core_axis_name="c", subcore_axis_name="s")

    @functools.partial(
        pl.kernel,
        mesh=mesh,
        compiler_params=pltpu.CompilerParams(needs_layout_passes=False),
        out_type=jax.ShapeDtypeStruct((BATCH, N_CLS), jnp.float32),
        scratch_types=[
            pltpu.VMEM((rows_per_w,), jnp.int32),
            pltpu.VMEM((ROWS_PER_CHUNK, N_CLS), jnp.float32),
            pltpu.VMEM((ROWS_PER_CHUNK, N_CLS), jnp.float32),
            pltpu.VMEM((ROWS_PER_CHUNK, N_CLS), jnp.float32),
            pltpu.SemaphoreType.DMA,
            pltpu.SemaphoreType.DMA,
            pltpu.SemaphoreType.DMA,
        ],
    )
    def k(x_hbm, zeros_hbm, out_hbm, idx_v, buf0, buf1, buf2, s0, s1, s2):
        wid = lax.axis_index("s") * info.num_cores + lax.axis_index("c")
        row_base = wid * rows_per_w

        pltpu.sync_copy(x_hbm.at[pl.ds(row_base, rows_per_w)], idx_v)
        pltpu.sync_copy(zeros_hbm, buf0)
        pltpu.sync_copy(zeros_hbm, buf1)
        pltpu.sync_copy(zeros_hbm, buf2)

        lane = lax.iota(jnp.int32, 16)
        ones16 = jnp.full((16,), 1.0, jnp.float32)
        zeros16 = jnp.zeros((16,), jnp.float32)
        bufs = (buf0, buf1, buf2)
        sems = (s0, s1, s2)
        handles = [None] * NBUF

        def scatter(buf, chunk, vals):
            for g in range(ROWS_PER_CHUNK // 16):
                x16 = idx_v[pl.ds(chunk * ROWS_PER_CHUNK + g * 16, 16)]
                plsc.store_scatter(buf, [g * 16 + lane, x16], vals)

        for c in range(n_chunks):
            b = c % NBUF
            if handles[b] is not None:
                handles[b].wait()
                scatter(bufs[b], c - NBUF, zeros16)
            scatter(bufs[b], c, ones16)
            dst = out_hbm.at[pl.ds(row_base + c * ROWS_PER_CHUNK, ROWS_PER_CHUNK)]
            handles[b] = pltpu.async_copy(bufs[b], dst, sems[b])

        for h in handles:
            h.wait()

    return k


_onehot = _build_onehot()


def kernel(X, eye):
    del eye
    zeros_blk = jnp.zeros((ROWS_PER_CHUNK, N_CLS), jnp.float32)
    return _onehot(X, zeros_blk)


# speedup vs baseline: 3.2182x; 2.1317x over previous
import functools

import jax
import jax.numpy as jnp
from jax import lax
from jax.experimental import pallas as pl
from jax.experimental.pallas import tpu as pltpu
from jax.experimental.pallas import tpu_sc as plsc

N_CLS = 1000
BATCH = 16384
COLS_PER_CHUNK = 128
NBUF = 1


def _build_onehot_t():
    info = plsc.get_sparse_core_info()
    nw = info.num_cores * info.num_subcores
    cols_per_w = BATCH // nw
    n_chunks = cols_per_w // COLS_PER_CHUNK
    mesh = plsc.VectorSubcoreMesh(core_axis_name="c", subcore_axis_name="s")

    @functools.partial(
        pl.kernel,
        mesh=mesh,
        compiler_params=pltpu.CompilerParams(needs_layout_passes=False),
        out_type=jax.ShapeDtypeStruct((N_CLS, BATCH), jnp.float32),
        scratch_types=[
            pltpu.VMEM((cols_per_w,), jnp.int32),
            pltpu.VMEM((N_CLS, COLS_PER_CHUNK), jnp.float32),
            pltpu.SemaphoreType.DMA,
        ],
    )
    def k(x_hbm, zeros_hbm, out_hbm, idx_v, buf0, s0):
        wid = lax.axis_index("s") * info.num_cores + lax.axis_index("c")
        col_base = wid * cols_per_w

        pltpu.sync_copy(x_hbm.at[pl.ds(col_base, cols_per_w)], idx_v)
        pltpu.sync_copy(zeros_hbm, buf0)

        lane = lax.iota(jnp.int32, 16)
        ones16 = jnp.full((16,), 1.0, jnp.float32)
        zeros16 = jnp.zeros((16,), jnp.float32)
        bufs = (buf0,)
        sems = (s0,)
        handles = [None] * NBUF

        def scatter(buf, chunk, vals):
            for g in range(COLS_PER_CHUNK // 16):
                x16 = idx_v[pl.ds(chunk * COLS_PER_CHUNK + g * 16, 16)]
                plsc.store_scatter(buf, [x16, g * 16 + lane], vals)

        for c in range(n_chunks):
            b = c % NBUF
            if handles[b] is not None:
                handles[b].wait()
                scatter(bufs[b], c - NBUF, zeros16)
            scatter(bufs[b], c, ones16)
            dst = out_hbm.at[:, pl.ds(col_base + c * COLS_PER_CHUNK, COLS_PER_CHUNK)]
            handles[b] = pltpu.async_copy(bufs[b], dst, sems[b])

        for h in handles:
            h.wait()

    return k


_onehot_t = _build_onehot_t()


def kernel(X, eye):
    del eye
    zeros_blk = jnp.zeros((N_CLS, COLS_PER_CHUNK), jnp.float32)
    return _onehot_t(X, zeros_blk).T
